# baseline (speedup 1.0000x reference)
import functools

import jax
import jax.numpy as jnp
from jax.experimental import pallas as pl

SMOOTHING_VALUE = 0.1
CONFIDENCE_VALUE = 1.0 - SMOOTHING_VALUE
PADDING_IDX = 0
TGT_VOCAB_SIZE = 32000
BATCH = 4096

ROW_BLOCK = 128


def _fill_body(tgt_ref, out_ref):
    eps = SMOOTHING_VALUE / (TGT_VOCAB_SIZE - 2)
    t = tgt_ref[...]
    cols = jax.lax.broadcasted_iota(jnp.int32, (ROW_BLOCK, TGT_VOCAB_SIZE), 1)
    body = jnp.where(cols == t, CONFIDENCE_VALUE,
                     jnp.where(cols == PADDING_IDX, 0.0, eps))
    out_ref[...] = jnp.where(t == PADDING_IDX, 0.0, body)


@functools.partial(jax.jit, static_argnames=("interpret",))
def kernel(tgt_token_ids_batch, interpret=False):
    b = tgt_token_ids_batch.shape[0]
    grid = (b // ROW_BLOCK,)
    return pl.pallas_call(
        _fill_body,
        grid=grid,
        in_specs=[pl.BlockSpec((ROW_BLOCK, 1), lambda i: (i, 0))],
        out_specs=pl.BlockSpec((ROW_BLOCK, TGT_VOCAB_SIZE), lambda i: (i, 0)),
        out_shape=jax.ShapeDtypeStruct((b, TGT_VOCAB_SIZE), jnp.float32),
        interpret=interpret,
    )(tgt_token_ids_batch)


# speedup vs baseline: 1.0188x; 1.0188x over previous
import functools

import jax
import jax.numpy as jnp
from jax.experimental import pallas as pl

SMOOTHING_VALUE = 0.1
CONFIDENCE_VALUE = 1.0 - SMOOTHING_VALUE
PADDING_IDX = 0
TGT_VOCAB_SIZE = 32000
BATCH = 4096

ROW_BLOCK = 96


def _fill_body(tgt_ref, out_ref):
    eps = SMOOTHING_VALUE / (TGT_VOCAB_SIZE - 2)
    t = tgt_ref[...]
    cols = jax.lax.broadcasted_iota(jnp.int32, (ROW_BLOCK, TGT_VOCAB_SIZE), 1)
    body = jnp.where(cols == t, CONFIDENCE_VALUE,
                     jnp.where(cols == PADDING_IDX, 0.0, eps))
    out_ref[...] = jnp.where(t == PADDING_IDX, 0.0, body)


@functools.partial(jax.jit, static_argnames=("interpret",))
def kernel(tgt_token_ids_batch, interpret=False):
    b = tgt_token_ids_batch.shape[0]
    grid = (b // ROW_BLOCK,)
    return pl.pallas_call(
        _fill_body,
        grid=grid,
        in_specs=[pl.BlockSpec((ROW_BLOCK, 1), lambda i: (i, 0))],
        out_specs=pl.BlockSpec((ROW_BLOCK, TGT_VOCAB_SIZE), lambda i: (i, 0)),
        out_shape=jax.ShapeDtypeStruct((b, TGT_VOCAB_SIZE), jnp.float32),
        interpret=interpret,
    )(tgt_token_ids_batch)


# speedup vs baseline: 1.0200x; 1.0012x over previous
import functools

import jax
import jax.numpy as jnp
from jax.experimental import pallas as pl

SMOOTHING_VALUE = 0.1
CONFIDENCE_VALUE = 1.0 - SMOOTHING_VALUE
PADDING_IDX = 0
TGT_VOCAB_SIZE = 32000
BATCH = 4096

ROW_BLOCK = 112


def _fill_body(tgt_ref, out_ref):
    eps = SMOOTHING_VALUE / (TGT_VOCAB_SIZE - 2)
    t = tgt_ref[...]
    cols = jax.lax.broadcasted_iota(jnp.int32, (ROW_BLOCK, TGT_VOCAB_SIZE), 1)
    body = jnp.where(cols == t, CONFIDENCE_VALUE,
                     jnp.where(cols == PADDING_IDX, 0.0, eps))
    out_ref[...] = jnp.where(t == PADDING_IDX, 0.0, body)


@functools.partial(jax.jit, static_argnames=("interpret",))
def kernel(tgt_token_ids_batch, interpret=False):
    b = tgt_token_ids_batch.shape[0]
    grid = (b // ROW_BLOCK,)
    return pl.pallas_call(
        _fill_body,
        grid=grid,
        in_specs=[pl.BlockSpec((ROW_BLOCK, 1), lambda i: (i, 0))],
        out_specs=pl.BlockSpec((ROW_BLOCK, TGT_VOCAB_SIZE), lambda i: (i, 0)),
        out_shape=jax.ShapeDtypeStruct((b, TGT_VOCAB_SIZE), jnp.float32),
        interpret=interpret,
    )(tgt_token_ids_batch)
